# baseline (speedup 1.0000x reference)
import jax
import jax.numpy as jnp
from jax.experimental import pallas as pl
from jax.experimental.pallas import tpu as pltpu

POP_ = 128
GENE_ = 4096
BJ_ = 512
NSTEPS_ = GENE_ // BJ_
NBUF_ = 3


def _quadform_kernel(x_ref, q_hbm, out_ref, b0, b1, b2, acc_ref, sems):
    j = pl.program_id(0)
    bufs = (b0, b1, b2)

    def copy(block, t):
        return pltpu.make_async_copy(
            q_hbm.at[pl.ds(block * BJ_, BJ_), :], bufs[t], sems.at[t])

    @pl.when(j == 0)
    def _prologue():
        copy(0, 0).start()
        copy(1, 1).start()
        acc_ref[...] = jnp.zeros_like(acc_ref)

    for t in range(NBUF_):
        nxt = j + NBUF_ - 1

        @pl.when((nxt % NBUF_ == t) & (nxt < NSTEPS_))
        def _issue(t=t, nxt=nxt):
            copy(nxt, t).start()

    for t in range(NBUF_):
        @pl.when(j % NBUF_ == t)
        def _compute(t=t):
            copy(j, t).wait()
            xj = x_ref[:, pl.ds(j * BJ_, BJ_)]
            y = jnp.dot(xj, bufs[t][...],
                        preferred_element_type=jnp.float32,
                        precision=jax.lax.Precision.DEFAULT)
            acc_ref[...] += y

    @pl.when(j == NSTEPS_ - 1)
    def _finish():
        out_ref[...] = jnp.sum(acc_ref[...] * x_ref[...], axis=1)[None, :]


@jax.jit
def kernel(keys_pop, Q):
    out = pl.pallas_call(
        _quadform_kernel,
        grid=(NSTEPS_,),
        in_specs=[
            pl.BlockSpec((POP_, GENE_), lambda j: (0, 0)),
            pl.BlockSpec(memory_space=pltpu.MemorySpace.HBM),
        ],
        out_specs=pl.BlockSpec((1, POP_), lambda j: (0, 0)),
        out_shape=jax.ShapeDtypeStruct((1, POP_), jnp.float32),
        scratch_shapes=[
            pltpu.VMEM((BJ_, GENE_), jnp.float32),
            pltpu.VMEM((BJ_, GENE_), jnp.float32),
            pltpu.VMEM((BJ_, GENE_), jnp.float32),
            pltpu.VMEM((POP_, GENE_), jnp.float32),
            pltpu.SemaphoreType.DMA((NBUF_,)),
        ],
    )(keys_pop, Q)
    return out[0]


# speedup vs baseline: 1.0867x; 1.0867x over previous
import jax
import jax.numpy as jnp
from jax.experimental import pallas as pl

POP_ = 128
GENE_ = 4096
BK_ = 512


def _quadform_kernel(x_ref, q_ref, out_ref):
    k = pl.program_id(0)
    x = x_ref[...]
    q = q_ref[...]
    y = jnp.dot(x, q, preferred_element_type=jnp.float32,
                precision=jax.lax.Precision.DEFAULT)
    xk = x_ref[:, pl.ds(k * BK_, BK_)]
    partial = jnp.sum(y * xk, axis=1)

    @pl.when(k == 0)
    def _init():
        out_ref[...] = partial[None, :]

    @pl.when(k > 0)
    def _acc():
        out_ref[...] += partial[None, :]


@jax.jit
def kernel(keys_pop, Q):
    out = pl.pallas_call(
        _quadform_kernel,
        grid=(GENE_ // BK_,),
        in_specs=[
            pl.BlockSpec((POP_, GENE_), lambda k: (0, 0)),
            pl.BlockSpec((GENE_, BK_), lambda k: (0, k)),
        ],
        out_specs=pl.BlockSpec((1, POP_), lambda k: (0, 0)),
        out_shape=jax.ShapeDtypeStruct((1, POP_), jnp.float32),
    )(keys_pop, Q)
    return out[0]
